# baseline (speedup 1.0000x reference)
import jax
import jax.numpy as jnp
from jax.experimental import pallas as pl

N = 10000
NC = 16
LPA_ITERS = 20


def _copy_body(x_ref, o_ref):
    o_ref[...] = x_ref[...]


def kernel(h, labels, train, edge_index, edge_weight, W1, b1, W2, b2):
    src = edge_index[0]
    dst = edge_index[1]
    ones = jnp.ones((src.shape[0],), dtype=jnp.float32)
    deg_in = jax.ops.segment_sum(ones, dst, num_segments=N)
    deg_out = jax.ops.segment_sum(ones, src, num_segments=N)
    tm = train[:, None]

    emb = jnp.where(tm, labels, 0.0)

    def lpa_step(e, _):
        m = e[src] * edge_weight
        s = jax.ops.segment_sum(m, dst, num_segments=N)
        e2 = jnp.where(deg_in[:, None] > 0, s / jnp.maximum(deg_in, 1.0)[:, None], 0.0)
        e2 = jnp.where(tm, labels, e2)
        return e2, None

    emb, _ = jax.lax.scan(lpa_step, emb, None, length=LPA_ITERS)

    ew = edge_weight
    nin = jnp.power(jnp.maximum(deg_in, 1.0), -0.5)[:, None]
    nout = jnp.power(jnp.maximum(deg_out, 1.0), -0.5)[:, None]

    def gcn(x, W, b):
        x = (x * nout) @ W
        s = jax.ops.segment_sum(x[src] * ew, dst, num_segments=N)
        return s * nin + b

    x1 = jax.nn.relu(gcn(h, W1, b1))
    logits = gcn(x1, W2, b2)

    out1 = jax.nn.log_softmax(logits, axis=1)
    out2 = jax.nn.log_softmax(emb, axis=1)
    out1 = pl.pallas_call(
        _copy_body, out_shape=jax.ShapeDtypeStruct(out1.shape, out1.dtype)
    )(out1)
    return (out1, out2)


# speedup vs baseline: 9.2462x; 9.2462x over previous
import functools

import jax
import jax.numpy as jnp
from jax.experimental import pallas as pl
from jax.experimental.pallas import tpu as pltpu

N = 10000
E = 320000
F = 128
NC = 16
ITERS = 20

NP = 10240
R = 256
NT = NP // R


def _prep_body(din_ref, dout_ref, hp_ref, lp_ref, tm_ref, w1_ref,
               invd_ref, nin_ref, lm_ref, ntm_ref, x_ref):
    din = din_ref[...]
    dout = dout_ref[...]
    invd_ref[...] = jnp.where(din > 0.0, 1.0 / jnp.maximum(din, 1.0), 0.0)
    nin_ref[...] = jax.lax.rsqrt(jnp.maximum(din, 1.0))
    nout = jax.lax.rsqrt(jnp.maximum(dout, 1.0))
    tm = tm_ref[...]
    lm_ref[...] = lp_ref[...] * tm
    ntm_ref[...] = 1.0 - tm
    x_ref[...] = jnp.dot(hp_ref[...] * nout, w1_ref[...],
                         preferred_element_type=jnp.float32)


def _lpa_body(m_ref, lmb_ref, ntmb_ref, invdb_ref, lmfull_ref, out_ref, ebuf):
    t = pl.program_id(0)
    i = pl.program_id(1)

    @pl.when(jnp.logical_and(t == 0, i == 0))
    def _():
        ebuf[0] = lmfull_ref[...]

    p = t % 2
    e = jnp.where(p == 0, ebuf[0], ebuf[1])
    s = jnp.dot(m_ref[...], e, preferred_element_type=jnp.float32)
    enew = lmb_ref[...] + ntmb_ref[...] * (s * invdb_ref[...])

    @pl.when(p == 0)
    def _():
        ebuf[1, pl.ds(i * R, R)] = enew

    @pl.when(p == 1)
    def _():
        ebuf[0, pl.ds(i * R, R)] = enew

    out_ref[...] = enew


def _gcn1_body(m_ref, x_ref, ninb_ref, noutb_ref, b1_ref, w2_ref, x2_ref):
    s = jnp.dot(m_ref[...], x_ref[...], preferred_element_type=jnp.float32)
    x1 = jnp.maximum(s * ninb_ref[...] + b1_ref[...], 0.0)
    x2_ref[...] = jnp.dot(x1 * noutb_ref[...], w2_ref[...],
                          preferred_element_type=jnp.float32)


def _logsoftmax(x):
    m = jnp.max(x, axis=1, keepdims=True)
    return x - m - jnp.log(jnp.sum(jnp.exp(x - m), axis=1, keepdims=True))


def _gcn2_body(m_ref, x2_ref, ninb_ref, b2_ref, embb_ref, o1_ref, o2_ref):
    s2 = jnp.dot(m_ref[...], x2_ref[...], preferred_element_type=jnp.float32)
    o1_ref[...] = _logsoftmax(s2 * ninb_ref[...] + b2_ref[...])
    o2_ref[...] = _logsoftmax(embb_ref[...])


@jax.jit
def kernel(h, labels, train, edge_index, edge_weight, W1, b1, W2, b2):
    src = edge_index[0]
    dst = edge_index[1]
    w = edge_weight[:, 0]

    M = jnp.zeros((NP, NP), jnp.float32).at[dst, src].add(w)
    din = jnp.zeros((NP,), jnp.float32).at[dst].add(1.0)[:, None]
    dout = jnp.zeros((NP,), jnp.float32).at[src].add(1.0)[:, None]
    hp = jnp.zeros((NP, F), jnp.float32).at[:N].set(h)
    lp = jnp.zeros((NP, NC), jnp.float32).at[:N].set(labels)
    tm = jnp.zeros((NP, 1), jnp.float32).at[:N, 0].set(train.astype(jnp.float32))

    full = lambda s: pl.BlockSpec(s, lambda *a: tuple(0 for _ in s))

    invd, nin, lm, ntm, x = pl.pallas_call(
        _prep_body,
        out_shape=[
            jax.ShapeDtypeStruct((NP, 1), jnp.float32),
            jax.ShapeDtypeStruct((NP, 1), jnp.float32),
            jax.ShapeDtypeStruct((NP, NC), jnp.float32),
            jax.ShapeDtypeStruct((NP, 1), jnp.float32),
            jax.ShapeDtypeStruct((NP, F), jnp.float32),
        ],
    )(din, dout, hp, lp, tm, W1)

    nout = jax.lax.rsqrt(jnp.maximum(dout, 1.0))

    row = lambda s: pl.BlockSpec(s, lambda t, i: (i, 0))
    fullg = lambda s: pl.BlockSpec(s, lambda t, i: (0, 0))
    emb = pl.pallas_call(
        _lpa_body,
        grid=(ITERS, NT),
        in_specs=[row((R, NP)), row((R, NC)), row((R, 1)), row((R, 1)),
                  fullg((NP, NC))],
        out_specs=row((R, NC)),
        out_shape=jax.ShapeDtypeStruct((NP, NC), jnp.float32),
        scratch_shapes=[pltpu.VMEM((2, NP, NC), jnp.float32)],
    )(M, lm, ntm, invd, lm)

    row1 = lambda s: pl.BlockSpec(s, lambda i: (i, 0))
    full1 = lambda s: pl.BlockSpec(s, lambda i: (0, 0))
    x2 = pl.pallas_call(
        _gcn1_body,
        grid=(NT,),
        in_specs=[row1((R, NP)), full1((NP, F)), row1((R, 1)), row1((R, 1)),
                  full1((1, F)), full1((F, NC))],
        out_specs=row1((R, NC)),
        out_shape=jax.ShapeDtypeStruct((NP, NC), jnp.float32),
    )(M, x, nin, nout, b1[None, :], W2)

    out1, out2 = pl.pallas_call(
        _gcn2_body,
        grid=(NT,),
        in_specs=[row1((R, NP)), full1((NP, NC)), row1((R, 1)),
                  full1((1, NC)), row1((R, NC))],
        out_specs=[row1((R, NC)), row1((R, NC))],
        out_shape=[jax.ShapeDtypeStruct((NP, NC), jnp.float32),
                   jax.ShapeDtypeStruct((NP, NC), jnp.float32)],
    )(M, x2, nin, b2[None, :], emb)

    return out1[:N], out2[:N]
